# baseline (speedup 1.0000x reference)
import jax
import jax.numpy as jnp
from jax.experimental import pallas as pl
from jax.experimental.pallas import tpu as pltpu

T = 8
BM = 256


def _body(lab_ref, x_ref, w_ref, b_ref, out_ref):
    t = pl.program_id(1)

    @pl.when(t == 0)
    def _():
        out_ref[...] = jnp.zeros_like(out_ref)

    mask = lab_ref[...] == t
    y = jax.lax.dot_general(
        x_ref[...], w_ref[0],
        dimension_numbers=(((1,), (1,)), ((), ())),
        preferred_element_type=jnp.float32,
    ) + b_ref[0]
    out_ref[...] += jnp.where(mask, y, 0.0)


def kernel(x, task_labels, W, b):
    B, D = x.shape
    OUT = W.shape[1]
    labs = task_labels.astype(jnp.int32).reshape(B, 1)
    b3 = b.reshape(T, 1, OUT)
    grid = (B // BM, T)
    return pl.pallas_call(
        _body,
        grid=grid,
        in_specs=[
            pl.BlockSpec((BM, 1), lambda i, t: (i, 0)),
            pl.BlockSpec((BM, D), lambda i, t: (i, 0)),
            pl.BlockSpec((1, OUT, D), lambda i, t: (t, 0, 0)),
            pl.BlockSpec((1, 1, OUT), lambda i, t: (t, 0, 0)),
        ],
        out_specs=pl.BlockSpec((BM, OUT), lambda i, t: (i, 0)),
        out_shape=jax.ShapeDtypeStruct((B, OUT), jnp.float32),
    )(labs, x, W, b3)


# speedup vs baseline: 1.1607x; 1.1607x over previous
import jax
import jax.numpy as jnp
from jax.experimental import pallas as pl
from jax.experimental.pallas import tpu as pltpu

T = 8
BM = 256


def _body(lab_ref, x_ref, w_ref, b_ref, out_ref):
    t = pl.program_id(1)

    @pl.when(t == 0)
    def _():
        out_ref[...] = jnp.zeros_like(out_ref)

    mask = lab_ref[...] == t
    y = jax.lax.dot_general(
        x_ref[...], w_ref[0],
        dimension_numbers=(((1,), (1,)), ((), ())),
        preferred_element_type=jnp.float32,
    ) + b_ref[0]
    out_ref[...] += jnp.where(mask, y, 0.0)


def kernel(x, task_labels, W, b):
    B, D = x.shape
    OUT = W.shape[1]
    labs = task_labels.astype(jnp.int32).reshape(B, 1)
    b3 = b.reshape(T, 1, OUT)
    x = x.astype(jnp.bfloat16)
    W = W.astype(jnp.bfloat16)
    grid = (B // BM, T)
    return pl.pallas_call(
        _body,
        grid=grid,
        in_specs=[
            pl.BlockSpec((BM, 1), lambda i, t: (i, 0)),
            pl.BlockSpec((BM, D), lambda i, t: (i, 0)),
            pl.BlockSpec((1, OUT, D), lambda i, t: (t, 0, 0)),
            pl.BlockSpec((1, 1, OUT), lambda i, t: (t, 0, 0)),
        ],
        out_specs=pl.BlockSpec((BM, OUT), lambda i, t: (i, 0)),
        out_shape=jax.ShapeDtypeStruct((B, OUT), jnp.float32),
    )(labs, x, W, b3)
